# Initial kernel scaffold; baseline (speedup 1.0000x reference)
import jax
import jax.numpy as jnp
from jax.experimental import pallas as pl


def kernel(feature, edge_index, W, b):
    raise NotImplementedError("write your pallas kernel here")


# speedup vs baseline: 3.0965x; 3.0965x over previous
import functools
import jax
import jax.numpy as jnp
from jax import lax
from jax.experimental import pallas as pl
from jax.experimental.pallas import tpu as pltpu
from jax.experimental.pallas import tpu_sc as plsc

N_NODES = 10000
D = 128
N_EDGES = 320000

NC = 2
NS = 16
NW = NC * NS

K = 128
STEPS = 80
CSTEPS = 16
E_PER_W = K * STEPS
E_PAD = NW * E_PER_W

N_ACC = 10240
ROWS_PER_TILE = N_ACC // NS


def _sc_body(feat_hbm, src_hbm, dst_hbm, zrow_hbm, ones_hbm,
             agg_hbm, deg_hbm,
             srci, dsti, rows, ones_v, agg_sh):
    cid = lax.axis_index("c")
    sid = lax.axis_index("s")
    wid = sid * NC + cid
    base = sid * ROWS_PER_TILE

    def zero_acc():
        pltpu.sync_copy(zrow_hbm, rows)
        for p in range(ROWS_PER_TILE // K):
            pltpu.sync_copy(rows, agg_sh.at[pl.ds(base + p * K, K)])

    def writeback(out_hbm):
        for p in range(ROWS_PER_TILE // K):
            pltpu.sync_copy(agg_sh.at[pl.ds(base + p * K, K)], rows)
            pltpu.sync_copy(rows, out_hbm.at[cid, pl.ds(base + p * K, K)])

    zero_acc()
    pltpu.sync_copy(ones_hbm, ones_v)
    plsc.subcore_barrier()

    def chunk(c, carry):
        off = wid * STEPS + c * CSTEPS
        pltpu.sync_copy(src_hbm.at[pl.ds(off, CSTEPS)], srci)
        pltpu.sync_copy(dst_hbm.at[pl.ds(off, CSTEPS)], dsti)

        def step(j, carry2):
            pltpu.sync_copy(feat_hbm.at[srci.at[j]], rows)
            pltpu.sync_copy(rows, agg_sh.at[dsti.at[j]], add=True)
            return carry2

        lax.fori_loop(0, CSTEPS, step, 0)
        return carry

    lax.fori_loop(0, STEPS // CSTEPS, chunk, 0)

    plsc.subcore_barrier()
    writeback(agg_hbm)

    plsc.subcore_barrier()
    zero_acc()
    plsc.subcore_barrier()

    def chunk2(c, carry):
        off = wid * STEPS + c * CSTEPS
        pltpu.sync_copy(dst_hbm.at[pl.ds(off, CSTEPS)], dsti)

        def step(j, carry2):
            pltpu.sync_copy(ones_v, agg_sh.at[dsti.at[j]], add=True)
            return carry2

        lax.fori_loop(0, CSTEPS, step, 0)
        return carry

    lax.fori_loop(0, STEPS // CSTEPS, chunk2, 0)

    plsc.subcore_barrier()
    writeback(deg_hbm)


_sc_aggregate = functools.partial(
    pl.kernel,
    _sc_body,
    out_type=(
        jax.ShapeDtypeStruct((NC, N_ACC, D), jnp.float32),
        jax.ShapeDtypeStruct((NC, N_ACC, D), jnp.float32),
    ),
    mesh=plsc.VectorSubcoreMesh(core_axis_name="c", subcore_axis_name="s"),
    scratch_types=[
        pltpu.VMEM((CSTEPS, K), jnp.int32),
        pltpu.VMEM((CSTEPS, K), jnp.int32),
        pltpu.VMEM((K, D), jnp.float32),
        pltpu.VMEM((K, D), jnp.float32),
        pltpu.VMEM_SHARED((N_ACC, D), jnp.float32),
    ],
)()


ROW_BLK = 1000


def _tc_body(feat, agg0, agg1, deg0, deg1, w, b, out):
    a = agg0[...] + agg1[...]
    d = deg0[...][:, 0:1] + deg1[...][:, 0:1]
    mean = a / jnp.maximum(d, 1.0)
    h = jnp.where(d > 0.0, mean, feat[...])
    acc = lax.dot_general(h, w[...], (((1,), (1,)), ((), ())),
                          preferred_element_type=jnp.float32)
    out[...] = jnp.maximum(acc + b[...], 0.0)


def _tc_update(feat, agg0, agg1, deg0, deg1, w, b2d):
    grid = N_NODES // ROW_BLK
    blk = pl.BlockSpec((ROW_BLK, D), lambda i: (i, 0))
    return pl.pallas_call(
        _tc_body,
        grid=(grid,),
        in_specs=[
            blk,
            blk,
            blk,
            blk,
            blk,
            pl.BlockSpec((D, D), lambda i: (0, 0)),
            pl.BlockSpec((1, D), lambda i: (0, 0)),
        ],
        out_specs=blk,
        out_shape=jax.ShapeDtypeStruct((N_NODES, D), jnp.float32),
    )(feat, agg0, agg1, deg0, deg1, w, b2d)


@jax.jit
def kernel(feature, edge_index, W, b):
    src = edge_index[0].astype(jnp.int32)
    dst = edge_index[1].astype(jnp.int32)
    pad = E_PAD - N_EDGES
    src_p = jnp.concatenate([src, jnp.zeros((pad,), jnp.int32)]).reshape(-1, K)
    dst_p = jnp.concatenate(
        [dst, jnp.full((pad,), N_NODES, jnp.int32)]).reshape(-1, K)

    zrow = jnp.zeros((K, D), jnp.float32)
    ones = jnp.ones((K, D), jnp.float32)

    agg, deg = _sc_aggregate(feature, src_p, dst_p, zrow, ones)

    return _tc_update(feature,
                      agg[0, :N_NODES], agg[1, :N_NODES],
                      deg[0, :N_NODES], deg[1, :N_NODES],
                      W, b.reshape(1, D))


# speedup vs baseline: 3.3093x; 1.0687x over previous
import functools
import jax
import jax.numpy as jnp
from jax import lax
from jax.experimental import pallas as pl
from jax.experimental.pallas import tpu as pltpu
from jax.experimental.pallas import tpu_sc as plsc

N_NODES = 10000
D = 128
N_EDGES = 320000

NC = 2
NS = 16
NW = NC * NS

K = 128
STEPS = 80
CSTEPS = 16
E_PER_W = K * STEPS
E_PAD = NW * E_PER_W

N_ACC = 10240
ROWS_PER_TILE = N_ACC // NS


def _sc_body(feat_hbm, src_hbm, dst_hbm, zrow_hbm, ones_hbm,
             agg_hbm, deg_hbm,
             srci, dsti, r0, r1, agg_sh, gsem, ssem):
    cid = lax.axis_index("c")
    sid = lax.axis_index("s")
    wid = sid * NC + cid
    base = sid * ROWS_PER_TILE

    def zero_acc():
        pltpu.sync_copy(zrow_hbm, r0)
        for p in range(ROWS_PER_TILE // K):
            pltpu.sync_copy(r0, agg_sh.at[pl.ds(base + p * K, K)])

    def writeback(out_hbm):
        for p in range(ROWS_PER_TILE // K):
            pltpu.sync_copy(agg_sh.at[pl.ds(base + p * K, K)], r0)
            pltpu.sync_copy(r0, out_hbm.at[cid, pl.ds(base + p * K, K)])

    def wait_gather(buf):
        pltpu.make_async_copy(feat_hbm.at[pl.ds(0, K)], buf, gsem).wait()

    zero_acc()
    plsc.subcore_barrier()

    def chunk(c, carry):
        off = wid * STEPS + c * CSTEPS
        pltpu.sync_copy(src_hbm.at[pl.ds(off, CSTEPS)], srci)
        pltpu.sync_copy(dst_hbm.at[pl.ds(off, CSTEPS)], dsti)

        pltpu.async_copy(feat_hbm.at[srci.at[0]], r0, gsem)

        def pair(p, carry2):
            j0 = 2 * p
            wait_gather(r0)
            pltpu.async_copy(feat_hbm.at[srci.at[j0 + 1]], r1, gsem)
            pltpu.sync_copy(r0, agg_sh.at[dsti.at[j0]], add=True)
            wait_gather(r1)

            @pl.when(p < CSTEPS // 2 - 1)
            def _():
                pltpu.async_copy(feat_hbm.at[srci.at[j0 + 2]], r0, gsem)

            pltpu.sync_copy(r1, agg_sh.at[dsti.at[j0 + 1]], add=True)
            return carry2

        lax.fori_loop(0, CSTEPS // 2, pair, 0)
        return carry

    lax.fori_loop(0, STEPS // CSTEPS, chunk, 0)

    plsc.subcore_barrier()
    writeback(agg_hbm)

    plsc.subcore_barrier()
    zero_acc()
    pltpu.sync_copy(ones_hbm, r1)
    plsc.subcore_barrier()

    def chunk2(c, carry):
        off = wid * STEPS + c * CSTEPS
        pltpu.sync_copy(dst_hbm.at[pl.ds(off, CSTEPS)], dsti)

        def fire(j, carry2):
            pltpu.async_copy(r1, agg_sh.at[dsti.at[j]], ssem, add=True)
            return carry2

        lax.fori_loop(0, CSTEPS, fire, 0)

        def drain(j, carry2):
            pltpu.make_async_copy(zrow_hbm, r1, ssem).wait()
            return carry2

        lax.fori_loop(0, CSTEPS, drain, 0)
        return carry

    lax.fori_loop(0, STEPS // CSTEPS, chunk2, 0)

    plsc.subcore_barrier()
    writeback(deg_hbm)


_sc_aggregate = functools.partial(
    pl.kernel,
    _sc_body,
    out_type=(
        jax.ShapeDtypeStruct((NC, N_ACC, D), jnp.float32),
        jax.ShapeDtypeStruct((NC, N_ACC, D), jnp.float32),
    ),
    mesh=plsc.VectorSubcoreMesh(core_axis_name="c", subcore_axis_name="s"),
    scratch_types=[
        pltpu.VMEM((CSTEPS, K), jnp.int32),
        pltpu.VMEM((CSTEPS, K), jnp.int32),
        pltpu.VMEM((K, D), jnp.float32),
        pltpu.VMEM((K, D), jnp.float32),
        pltpu.VMEM_SHARED((N_ACC, D), jnp.float32),
        pltpu.SemaphoreType.DMA,
        pltpu.SemaphoreType.DMA,
    ],
)()


ROW_BLK = 1000


def _tc_body(feat, agg0, agg1, deg0, deg1, w, b, out):
    a = agg0[...] + agg1[...]
    d = deg0[...][:, 0:1] + deg1[...][:, 0:1]
    mean = a / jnp.maximum(d, 1.0)
    h = jnp.where(d > 0.0, mean, feat[...])
    acc = lax.dot_general(h, w[...], (((1,), (1,)), ((), ())),
                          preferred_element_type=jnp.float32)
    out[...] = jnp.maximum(acc + b[...], 0.0)


def _tc_update(feat, agg0, agg1, deg0, deg1, w, b2d):
    grid = N_NODES // ROW_BLK
    blk = pl.BlockSpec((ROW_BLK, D), lambda i: (i, 0))
    return pl.pallas_call(
        _tc_body,
        grid=(grid,),
        in_specs=[
            blk,
            blk,
            blk,
            blk,
            blk,
            pl.BlockSpec((D, D), lambda i: (0, 0)),
            pl.BlockSpec((1, D), lambda i: (0, 0)),
        ],
        out_specs=blk,
        out_shape=jax.ShapeDtypeStruct((N_NODES, D), jnp.float32),
    )(feat, agg0, agg1, deg0, deg1, w, b2d)


@jax.jit
def kernel(feature, edge_index, W, b):
    src = edge_index[0].astype(jnp.int32)
    dst = edge_index[1].astype(jnp.int32)
    pad = E_PAD - N_EDGES
    src_p = jnp.concatenate([src, jnp.zeros((pad,), jnp.int32)]).reshape(-1, K)
    dst_p = jnp.concatenate(
        [dst, jnp.full((pad,), N_NODES, jnp.int32)]).reshape(-1, K)

    zrow = jnp.zeros((K, D), jnp.float32)
    ones = jnp.ones((K, D), jnp.float32)

    agg, deg = _sc_aggregate(feature, src_p, dst_p, zrow, ones)

    return _tc_update(feature,
                      agg[0, :N_NODES], agg[1, :N_NODES],
                      deg[0, :N_NODES], deg[1, :N_NODES],
                      W, b.reshape(1, D))


# speedup vs baseline: 8.4649x; 2.5579x over previous
import functools
import jax
import jax.numpy as jnp
from jax import lax
from jax.experimental import pallas as pl
from jax.experimental.pallas import tpu as pltpu
from jax.experimental.pallas import tpu_sc as plsc

N_NODES = 10000
D = 128
N_EDGES = 320000

NC = 2
NS = 16
NW = NC * NS

K = 128
STEPS = 80
CSTEPS = 16
E_PER_W = K * STEPS
E_PAD = NW * E_PER_W

N_ACC = 10240
ROWS_PER_TILE = N_ACC // NS


def _sc_body(feat_hbm, src_hbm, dst_hbm, zrow_hbm, ones_hbm,
             agg_hbm, deg_hbm,
             srci, dsti, r0, r1, agg_sh, gsem, ssem):
    cid = lax.axis_index("c")
    sid = lax.axis_index("s")
    wid = sid * NC + cid
    base = sid * ROWS_PER_TILE

    def zero_acc():
        pltpu.sync_copy(zrow_hbm, r0)
        for p in range(ROWS_PER_TILE // K):
            pltpu.sync_copy(r0, agg_sh.at[pl.ds(base + p * K, K)])

    def writeback(out_hbm):
        for p in range(ROWS_PER_TILE // K):
            pltpu.sync_copy(agg_sh.at[pl.ds(base + p * K, K)], r0)
            pltpu.sync_copy(r0, out_hbm.at[cid, pl.ds(base + p * K, K)])

    def wait_gather(buf):
        pltpu.make_async_copy(feat_hbm.at[pl.ds(0, K)], buf, gsem).wait()

    zero_acc()
    plsc.subcore_barrier()

    def chunk(c, carry):
        off = wid * STEPS + c * CSTEPS
        pltpu.sync_copy(src_hbm.at[pl.ds(off, CSTEPS)], srci)
        pltpu.sync_copy(dst_hbm.at[pl.ds(off, CSTEPS)], dsti)

        pltpu.async_copy(feat_hbm.at[srci.at[0]], r0, gsem)

        def pair(p, carry2):
            j0 = 2 * p
            wait_gather(r0)
            pltpu.async_copy(feat_hbm.at[srci.at[j0 + 1]], r1, gsem)
            pltpu.sync_copy(r0, agg_sh.at[dsti.at[j0]], add=True)
            wait_gather(r1)

            @pl.when(p < CSTEPS // 2 - 1)
            def _():
                pltpu.async_copy(feat_hbm.at[srci.at[j0 + 2]], r0, gsem)

            pltpu.sync_copy(r1, agg_sh.at[dsti.at[j0 + 1]], add=True)
            return carry2

        lax.fori_loop(0, CSTEPS // 2, pair, 0)
        return carry

    lax.fori_loop(0, STEPS // CSTEPS, chunk, 0)

    plsc.subcore_barrier()
    writeback(agg_hbm)

    plsc.subcore_barrier()
    zero_acc()
    pltpu.sync_copy(ones_hbm, r1)
    plsc.subcore_barrier()

    def chunk2(c, carry):
        off = wid * STEPS + c * CSTEPS
        pltpu.sync_copy(dst_hbm.at[pl.ds(off, CSTEPS)], dsti)

        def fire(j, carry2):
            pltpu.async_copy(r1, agg_sh.at[dsti.at[j]], ssem, add=True)
            return carry2

        lax.fori_loop(0, CSTEPS, fire, 0)

        def drain(j, carry2):
            pltpu.make_async_copy(zrow_hbm, r1, ssem).wait()
            return carry2

        lax.fori_loop(0, CSTEPS, drain, 0)
        return carry

    lax.fori_loop(0, STEPS // CSTEPS, chunk2, 0)

    plsc.subcore_barrier()
    writeback(deg_hbm)


_sc_aggregate = functools.partial(
    pl.kernel,
    _sc_body,
    out_type=(
        jax.ShapeDtypeStruct((NC, N_ACC, D), jnp.float32),
        jax.ShapeDtypeStruct((NC, N_ACC, D), jnp.float32),
    ),
    mesh=plsc.VectorSubcoreMesh(core_axis_name="c", subcore_axis_name="s"),
    scratch_types=[
        pltpu.VMEM((CSTEPS, K), jnp.int32),
        pltpu.VMEM((CSTEPS, K), jnp.int32),
        pltpu.VMEM((K, D), jnp.float32),
        pltpu.VMEM((K, D), jnp.float32),
        pltpu.VMEM_SHARED((N_ACC, D), jnp.float32),
        pltpu.SemaphoreType.DMA,
        pltpu.SemaphoreType.DMA,
    ],
)()


ROW_BLK = 1000


def _tc_body(feat, agg0, agg1, deg0, deg1, w, b, out):
    a = agg0[...] + agg1[...]
    d = deg0[...][:, 0:1] + deg1[...][:, 0:1]
    mean = a / jnp.maximum(d, 1.0)
    h = jnp.where(d > 0.0, mean, feat[...])
    acc = lax.dot_general(h, w[...], (((1,), (1,)), ((), ())),
                          preferred_element_type=jnp.float32)
    out[...] = jnp.maximum(acc + b[...], 0.0)


def _tc_update(feat, agg0, agg1, deg0, deg1, w, b2d):
    grid = N_NODES // ROW_BLK
    blk = pl.BlockSpec((ROW_BLK, D), lambda i: (i, 0))
    return pl.pallas_call(
        _tc_body,
        grid=(grid,),
        in_specs=[
            blk,
            blk,
            blk,
            blk,
            blk,
            pl.BlockSpec((D, D), lambda i: (0, 0)),
            pl.BlockSpec((1, D), lambda i: (0, 0)),
        ],
        out_specs=blk,
        out_shape=jax.ShapeDtypeStruct((N_NODES, D), jnp.float32),
    )(feat, agg0, agg1, deg0, deg1, w, b2d)


@jax.jit
def kernel(feature, edge_index, W, b):
    src = edge_index[0].astype(jnp.int32)
    dst = edge_index[1].astype(jnp.int32)
    pad = E_PAD - N_EDGES
    pad_iota = jnp.arange(pad, dtype=jnp.int32)
    src_p = jnp.concatenate([src, pad_iota % N_NODES]).reshape(-1, K)
    dst_p = jnp.concatenate(
        [dst, N_NODES + pad_iota % (N_ACC - N_NODES)]).reshape(-1, K)

    zrow = jnp.zeros((K, D), jnp.float32)
    ones = jnp.ones((K, D), jnp.float32)

    agg, deg = _sc_aggregate(feature, src_p, dst_p, zrow, ones)

    return _tc_update(feature,
                      agg[0, :N_NODES], agg[1, :N_NODES],
                      deg[0, :N_NODES], deg[1, :N_NODES],
                      W, b.reshape(1, D))


# speedup vs baseline: 9.0422x; 1.0682x over previous
import functools
import jax
import jax.numpy as jnp
from jax import lax
from jax.experimental import pallas as pl
from jax.experimental.pallas import tpu as pltpu
from jax.experimental.pallas import tpu_sc as plsc

N_NODES = 10000
D = 128
N_EDGES = 320000

NC = 2
NS = 16
NW = NC * NS

K = 128
STEPS = 80
CSTEPS = 16
E_PER_W = K * STEPS
E_PAD = NW * E_PER_W

N_ACC = 10240
ROWS_PER_TILE = N_ACC // NS


def _sc_body(feat_hbm, src_hbm, dst_hbm, zrow_hbm, ones_hbm,
             agg_hbm, deg_hbm,
             srci, dsti, r0, r1, agg_sh, gsem, ssem):
    cid = lax.axis_index("c")
    sid = lax.axis_index("s")
    wid = sid * NC + cid
    base = sid * ROWS_PER_TILE

    n_pieces = ROWS_PER_TILE // K
    sems = (gsem, ssem)
    bufs = (r0, r1)

    def zero_acc():
        pltpu.sync_copy(zrow_hbm, r0)
        for p in range(n_pieces):
            pltpu.async_copy(r0, agg_sh.at[pl.ds(base + p * K, K)], gsem)
        for p in range(n_pieces):
            pltpu.make_async_copy(zrow_hbm, r0, gsem).wait()

    def writeback(out_hbm):
        for p in range(min(2, n_pieces)):
            pltpu.async_copy(agg_sh.at[pl.ds(base + p * K, K)],
                             bufs[p % 2], sems[p % 2])
        for p in range(n_pieces):
            b, s = bufs[p % 2], sems[p % 2]
            pltpu.make_async_copy(zrow_hbm, b, s).wait()
            pltpu.async_copy(b, out_hbm.at[cid, pl.ds(base + p * K, K)], s)
            pltpu.make_async_copy(zrow_hbm, b, s).wait()
            if p + 2 < n_pieces:
                pltpu.async_copy(agg_sh.at[pl.ds(base + (p + 2) * K, K)],
                                 b, s)

    def wait_gather(buf):
        pltpu.make_async_copy(feat_hbm.at[pl.ds(0, K)], buf, gsem).wait()

    zero_acc()
    plsc.subcore_barrier()

    def chunk(c, carry):
        off = wid * STEPS + c * CSTEPS
        pltpu.sync_copy(src_hbm.at[pl.ds(off, CSTEPS)], srci)
        pltpu.sync_copy(dst_hbm.at[pl.ds(off, CSTEPS)], dsti)

        pltpu.async_copy(feat_hbm.at[srci.at[0]], r0, gsem)

        def pair(p, carry2):
            j0 = 2 * p
            wait_gather(r0)
            pltpu.async_copy(feat_hbm.at[srci.at[j0 + 1]], r1, gsem)
            pltpu.sync_copy(r0, agg_sh.at[dsti.at[j0]], add=True)
            wait_gather(r1)

            @pl.when(p < CSTEPS // 2 - 1)
            def _():
                pltpu.async_copy(feat_hbm.at[srci.at[j0 + 2]], r0, gsem)

            pltpu.sync_copy(r1, agg_sh.at[dsti.at[j0 + 1]], add=True)
            return carry2

        lax.fori_loop(0, CSTEPS // 2, pair, 0)
        return carry

    lax.fori_loop(0, STEPS // CSTEPS, chunk, 0)

    plsc.subcore_barrier()
    writeback(agg_hbm)

    plsc.subcore_barrier()
    zero_acc()
    pltpu.sync_copy(ones_hbm, r1)
    plsc.subcore_barrier()

    def chunk2(c, carry):
        off = wid * STEPS + c * CSTEPS
        pltpu.sync_copy(dst_hbm.at[pl.ds(off, CSTEPS)], dsti)

        def fire(j, carry2):
            pltpu.async_copy(r1, agg_sh.at[dsti.at[j]], ssem, add=True)
            return carry2

        lax.fori_loop(0, CSTEPS, fire, 0)

        def drain(j, carry2):
            pltpu.make_async_copy(zrow_hbm, r1, ssem).wait()
            return carry2

        lax.fori_loop(0, CSTEPS, drain, 0)
        return carry

    lax.fori_loop(0, STEPS // CSTEPS, chunk2, 0)

    plsc.subcore_barrier()
    writeback(deg_hbm)


_sc_aggregate = functools.partial(
    pl.kernel,
    _sc_body,
    out_type=(
        jax.ShapeDtypeStruct((NC, N_ACC, D), jnp.float32),
        jax.ShapeDtypeStruct((NC, N_ACC, D), jnp.float32),
    ),
    mesh=plsc.VectorSubcoreMesh(core_axis_name="c", subcore_axis_name="s"),
    scratch_types=[
        pltpu.VMEM((CSTEPS, K), jnp.int32),
        pltpu.VMEM((CSTEPS, K), jnp.int32),
        pltpu.VMEM((K, D), jnp.float32),
        pltpu.VMEM((K, D), jnp.float32),
        pltpu.VMEM_SHARED((N_ACC, D), jnp.float32),
        pltpu.SemaphoreType.DMA,
        pltpu.SemaphoreType.DMA,
    ],
)()


ROW_BLK = 1000


def _tc_body(feat, agg0, agg1, deg0, deg1, w, b, out):
    a = agg0[0] + agg1[0]
    d = deg0[0][:, 0:1] + deg1[0][:, 0:1]
    mean = a / jnp.maximum(d, 1.0)
    h = jnp.where(d > 0.0, mean, feat[...])
    acc = lax.dot_general(h, w[...], (((1,), (1,)), ((), ())),
                          preferred_element_type=jnp.float32)
    out[...] = jnp.maximum(acc + b[...], 0.0)


def _tc_update(feat, agg, deg, w, b2d):
    grid = N_NODES // ROW_BLK
    blk = pl.BlockSpec((ROW_BLK, D), lambda i: (i, 0))
    c0 = pl.BlockSpec((1, ROW_BLK, D), lambda i: (0, i, 0))
    c1 = pl.BlockSpec((1, ROW_BLK, D), lambda i: (1, i, 0))
    return pl.pallas_call(
        _tc_body,
        grid=(grid,),
        in_specs=[
            blk,
            c0,
            c1,
            c0,
            c1,
            pl.BlockSpec((D, D), lambda i: (0, 0)),
            pl.BlockSpec((1, D), lambda i: (0, 0)),
        ],
        out_specs=blk,
        out_shape=jax.ShapeDtypeStruct((N_NODES, D), jnp.float32),
    )(feat, agg, agg, deg, deg, w, b2d)


@jax.jit
def kernel(feature, edge_index, W, b):
    src = edge_index[0].astype(jnp.int32)
    dst = edge_index[1].astype(jnp.int32)
    pad = E_PAD - N_EDGES
    pad_iota = jnp.arange(pad, dtype=jnp.int32)
    src_p = jnp.concatenate([src, pad_iota % N_NODES]).reshape(-1, K)
    dst_p = jnp.concatenate(
        [dst, N_NODES + pad_iota % (N_ACC - N_NODES)]).reshape(-1, K)

    zrow = jnp.zeros((K, D), jnp.float32)
    ones = jnp.ones((K, D), jnp.float32)

    agg, deg = _sc_aggregate(feature, src_p, dst_p, zrow, ones)

    return _tc_update(feature, agg, deg, W, b.reshape(1, D))


# speedup vs baseline: 9.2238x; 1.0201x over previous
import functools
import jax
import jax.numpy as jnp
from jax import lax
from jax.experimental import pallas as pl
from jax.experimental.pallas import tpu as pltpu
from jax.experimental.pallas import tpu_sc as plsc

N_NODES = 10000
D = 128
N_EDGES = 320000

NC = 2
NS = 16
NW = NC * NS

K = 128
STEPS = 80
CSTEPS = 16
E_PER_W = K * STEPS
E_PAD = NW * E_PER_W

N_ACC = 10240
ROWS_PER_TILE = N_ACC // NS


def _sc_body(feat_hbm, src_hbm, dst_hbm, zrow_hbm, ones_hbm,
             agg_hbm, deg_hbm,
             srci, dsti, r0, r1, agg_sh, gsem, ssem):
    cid = lax.axis_index("c")
    sid = lax.axis_index("s")
    wid = sid * NC + cid
    base = sid * ROWS_PER_TILE

    n_pieces = ROWS_PER_TILE // K
    sems = (gsem, ssem)
    bufs = (r0, r1)

    def zero_acc():
        pltpu.sync_copy(zrow_hbm, r0)
        for p in range(n_pieces):
            pltpu.async_copy(r0, agg_sh.at[pl.ds(base + p * K, K)], gsem)
        for p in range(n_pieces):
            pltpu.make_async_copy(zrow_hbm, r0, gsem).wait()

    def writeback(out_hbm):
        for p in range(min(2, n_pieces)):
            pltpu.async_copy(agg_sh.at[pl.ds(base + p * K, K)],
                             bufs[p % 2], sems[p % 2])
        for p in range(n_pieces):
            b, s = bufs[p % 2], sems[p % 2]
            pltpu.make_async_copy(zrow_hbm, b, s).wait()
            pltpu.async_copy(b, out_hbm.at[cid, pl.ds(base + p * K, K)], s)
            pltpu.make_async_copy(zrow_hbm, b, s).wait()
            if p + 2 < n_pieces:
                pltpu.async_copy(agg_sh.at[pl.ds(base + (p + 2) * K, K)],
                                 b, s)

    def wait_gather(buf):
        pltpu.make_async_copy(feat_hbm.at[pl.ds(0, K)], buf, gsem).wait()

    zero_acc()
    plsc.subcore_barrier()

    def chunk(c, carry):
        off = wid * STEPS + c * CSTEPS
        pltpu.sync_copy(src_hbm.at[pl.ds(off, CSTEPS)], srci)
        pltpu.sync_copy(dst_hbm.at[pl.ds(off, CSTEPS)], dsti)

        pltpu.async_copy(feat_hbm.at[srci.at[0]], r0, gsem)

        def pair(p, carry2):
            j0 = 2 * p
            wait_gather(r0)
            pltpu.async_copy(feat_hbm.at[srci.at[j0 + 1]], r1, gsem)
            pltpu.sync_copy(r0, agg_sh.at[dsti.at[j0]], add=True)
            wait_gather(r1)

            @pl.when(p < CSTEPS // 2 - 1)
            def _():
                pltpu.async_copy(feat_hbm.at[srci.at[j0 + 2]], r0, gsem)

            pltpu.sync_copy(r1, agg_sh.at[dsti.at[j0 + 1]], add=True)
            return carry2

        lax.fori_loop(0, CSTEPS // 2, pair, 0)
        return carry

    lax.fori_loop(0, STEPS // CSTEPS, chunk, 0)

    plsc.subcore_barrier()
    writeback(agg_hbm)

    pltpu.sync_copy(ones_hbm, r1)
    plsc.subcore_barrier()

    def chunk2(c, carry):
        off = wid * STEPS + c * CSTEPS
        pltpu.sync_copy(dst_hbm.at[pl.ds(off, CSTEPS)], dsti)

        def fire(j, carry2):
            pltpu.async_copy(r1, agg_sh.at[dsti.at[j]], ssem, add=True)
            return carry2

        lax.fori_loop(0, CSTEPS, fire, 0)

        def drain(j, carry2):
            pltpu.make_async_copy(zrow_hbm, r1, ssem).wait()
            return carry2

        lax.fori_loop(0, CSTEPS, drain, 0)
        return carry

    lax.fori_loop(0, STEPS // CSTEPS, chunk2, 0)

    plsc.subcore_barrier()
    writeback(deg_hbm)


_sc_aggregate = functools.partial(
    pl.kernel,
    _sc_body,
    out_type=(
        jax.ShapeDtypeStruct((NC, N_ACC, D), jnp.float32),
        jax.ShapeDtypeStruct((NC, N_ACC, D), jnp.float32),
    ),
    mesh=plsc.VectorSubcoreMesh(core_axis_name="c", subcore_axis_name="s"),
    scratch_types=[
        pltpu.VMEM((CSTEPS, K), jnp.int32),
        pltpu.VMEM((CSTEPS, K), jnp.int32),
        pltpu.VMEM((K, D), jnp.float32),
        pltpu.VMEM((K, D), jnp.float32),
        pltpu.VMEM_SHARED((N_ACC, D), jnp.float32),
        pltpu.SemaphoreType.DMA,
        pltpu.SemaphoreType.DMA,
    ],
)()


ROW_BLK = 1000


def _tc_body(feat, agg0, agg1, deg0, deg1, w, b, out):
    a = agg0[0] + agg1[0]
    d = (deg0[0][:, 0:1] + deg1[0][:, 0:1]) - (agg0[0][:, 0:1]
                                               + agg1[0][:, 0:1])
    mean = a / jnp.maximum(d, 1.0)
    h = jnp.where(d > 0.0, mean, feat[...])
    acc = lax.dot_general(h, w[...], (((1,), (1,)), ((), ())),
                          preferred_element_type=jnp.float32)
    out[...] = jnp.maximum(acc + b[...], 0.0)


def _tc_update(feat, agg, deg, w, b2d):
    grid = N_NODES // ROW_BLK
    blk = pl.BlockSpec((ROW_BLK, D), lambda i: (i, 0))
    c0 = pl.BlockSpec((1, ROW_BLK, D), lambda i: (0, i, 0))
    c1 = pl.BlockSpec((1, ROW_BLK, D), lambda i: (1, i, 0))
    return pl.pallas_call(
        _tc_body,
        grid=(grid,),
        in_specs=[
            blk,
            c0,
            c1,
            c0,
            c1,
            pl.BlockSpec((D, D), lambda i: (0, 0)),
            pl.BlockSpec((1, D), lambda i: (0, 0)),
        ],
        out_specs=blk,
        out_shape=jax.ShapeDtypeStruct((N_NODES, D), jnp.float32),
    )(feat, agg, agg, deg, deg, w, b2d)


@jax.jit
def kernel(feature, edge_index, W, b):
    src = edge_index[0].astype(jnp.int32)
    dst = edge_index[1].astype(jnp.int32)
    pad = E_PAD - N_EDGES
    pad_iota = jnp.arange(pad, dtype=jnp.int32)
    src_p = jnp.concatenate([src, pad_iota % N_NODES]).reshape(-1, K)
    dst_p = jnp.concatenate(
        [dst, N_NODES + pad_iota % (N_ACC - N_NODES)]).reshape(-1, K)

    zrow = jnp.zeros((K, D), jnp.float32)
    ones = jnp.ones((K, D), jnp.float32)

    agg, deg = _sc_aggregate(feature, src_p, dst_p, zrow, ones)

    return _tc_update(feature, agg, deg, W, b.reshape(1, D))
